# baseline (speedup 1.0000x reference)
import functools

import jax
import jax.numpy as jnp
from jax import lax
from jax.experimental import pallas as pl
from jax.experimental.pallas import tpu as pltpu
from jax.experimental.pallas import tpu_sc as plsc

_B, _L, _D = 1024, 200, 128
_NC, _NS = 2, 16
_NW = _NC * _NS
_RW = _B // _NW
_NCH = (_L + 15) // 16
_LP = _NCH * 16
_TAIL = _L - (_NCH - 1) * 16
_VOCAB = 1000
_MAX_SEQ = 256

_mesh = plsc.VectorSubcoreMesh(core_axis_name="c", subcore_axis_name="s")


@functools.partial(
    pl.kernel,
    out_type=jax.ShapeDtypeStruct((_B, _L, _D), jnp.float32),
    mesh=_mesh,
    scratch_types=[
        pltpu.VMEM((_LP,), jnp.int32),
        pltpu.VMEM((_LP,), jnp.int32),
        pltpu.VMEM((_LP,), jnp.int32),
        pltpu.VMEM((_LP,), jnp.int32),
        pltpu.VMEM((_LP, _D), jnp.float32),
        pltpu.VMEM((_LP, _D), jnp.float32),
        pltpu.VMEM((_MAX_SEQ, _D), jnp.float32),
        pltpu.VMEM_SHARED((_VOCAB, _D), jnp.float32),
        pltpu.SemaphoreType.DMA,
        pltpu.SemaphoreType.DMA,
        pltpu.SemaphoreType.DMA,
        pltpu.SemaphoreType.DMA,
        pltpu.SemaphoreType.DMA,
        pltpu.SemaphoreType.DMA,
        pltpu.SemaphoreType.DMA,
        pltpu.SemaphoreType.DMA,
        pltpu.SemaphoreType.DMA,
        pltpu.SemaphoreType.DMA,
    ],
    compiler_params=pltpu.CompilerParams(needs_layout_passes=False),
)
def _embed_kernel(ids_hbm, char_hbm, pos_hbm, out_hbm,
                  idx0, idx1, pidx0, pidx1, cbuf0, cbuf1, ptab_v,
                  ctab_s,
                  sem_i0, sem_i1, sem_g0, sem_g1, sem_h0, sem_h1,
                  sem_p0, sem_p1, sem_o0, sem_o1):
    idxs = (idx0, idx1)
    pidxs = (pidx0, pidx1)
    cbufs = (cbuf0, cbuf1)
    sem_i = (sem_i0, sem_i1)
    sem_g = (sem_g0, sem_g1)
    sem_h = (sem_h0, sem_h1)
    sem_o = (sem_o0, sem_o1)

    wid = lax.axis_index("s") * _NC + lax.axis_index("c")
    row0 = wid * _RW
    lanes = lax.iota(jnp.int32, 16)
    ones = jnp.full((16,), 1, jnp.int32)
    zeros = jnp.full((16,), 0, jnp.int32)

    @pl.when(lax.axis_index("s") == 0)
    def _stage_ctab():
        pltpu.sync_copy(char_hbm, ctab_s)

    pltpu.sync_copy(pos_hbm, ptab_v)
    plsc.subcore_barrier()

    def issue_ids(r, b):
        return pltpu.async_copy(ids_hbm.at[pl.ds(r * _L, _L)],
                                idxs[b].at[pl.ds(0, _L)], sem_i[b])

    def wait_ids(b):
        pltpu.make_async_copy(ids_hbm.at[pl.ds(0, _L)],
                              idxs[b].at[pl.ds(0, _L)], sem_i[b]).wait()

    def pos_and_gather(b):
        carry = zeros
        for j in range(_NCH):
            v = idxs[b][pl.ds(j * 16, 16)]
            if j == _NCH - 1:
                v = jnp.where(lanes < _TAIL, v, zeros)
            if j == _NCH - 1:
                idxs[b][pl.ds(j * 16, 16)] = v
            m = jnp.where(v != 0, ones, zeros)
            pidxs[b][pl.ds(j * 16, 16)] = (plsc.cumsum(m) + carry) * m
            carry = carry + lax.broadcast_in_dim(jnp.sum(m), (16,), ())
        for lo, n, sem in ((0, 128, sem_g), (128, _LP - 128, sem_h)):
            sl = pl.ds(lo, n)
            pltpu.async_copy(ctab_s.at[idxs[b].at[sl]], cbufs[b].at[sl],
                             sem[b])

    def wait_gathers(b):
        pltpu.make_async_copy(char_hbm.at[pl.ds(0, 128)],
                              cbufs[b].at[pl.ds(0, 128)], sem_g[b]).wait()
        pltpu.make_async_copy(char_hbm.at[pl.ds(128, _LP - 128)],
                              cbufs[b].at[pl.ds(128, _LP - 128)],
                              sem_h[b]).wait()

    def wait_out(b, row):
        pltpu.make_async_copy(cbufs[b].at[pl.ds(0, _L)],
                              out_hbm.at[row], sem_o[b]).wait()

    issue_ids(row0, 0).wait()
    issue_ids(row0 + 1, 1)
    pos_and_gather(0)

    def outer(o, acc):
        for par in range(2):
            i = o * 2 + par
            cur, nxt = par, 1 - par
            row = row0 + i

            @pl.when(i < _RW - 1)
            def _prep_next():
                wait_ids(nxt)

                @pl.when(i > 0)
                def _wait_prev_out():
                    wait_out(nxt, row)

                pos_and_gather(nxt)

            wait_gathers(cur)

            @pl.when(i < _RW - 2)
            def _issue_ids2():
                issue_ids(row + 2, cur)

            def add_body(j, a):
                pv = pidxs[cur][pl.ds(j * 16, 16)]
                tv = j * 16 + lanes
                for c in range(_D):
                    cv = jnp.full((16,), c, jnp.int32)
                    g = plsc.load_gather(ptab_v, [pv, cv])
                    plsc.addupdate_scatter(cbufs[cur], [tv, cv], g)
                return a

            lax.fori_loop(0, _NCH, add_body, 0)
            pltpu.async_copy(cbufs[cur].at[pl.ds(0, _L)], out_hbm.at[row],
                             sem_o[cur])
        return acc

    lax.fori_loop(0, _RW // 2, outer, 0)
    wait_out(0, row0 + _RW - 2)
    wait_out(1, row0 + _RW - 1)


def kernel(input_ids, char_table, pos_table):
    return _embed_kernel(input_ids.reshape(-1), char_table, pos_table)


# speedup vs baseline: 2.0497x; 2.0497x over previous
import functools

import jax
import jax.numpy as jnp
from jax import lax
from jax.experimental import pallas as pl
from jax.experimental.pallas import tpu as pltpu
from jax.experimental.pallas import tpu_sc as plsc

_B, _L, _D = 1024, 200, 128
_NC, _NS = 2, 16
_NW = _NC * _NS
_RW = _B // _NW
_NCH = (_L + 15) // 16
_LP = _NCH * 16
_TAIL = _L - (_NCH - 1) * 16
_VOCAB = 1000

_mesh = plsc.VectorSubcoreMesh(core_axis_name="c", subcore_axis_name="s")


@functools.partial(
    pl.kernel,
    out_type=jax.ShapeDtypeStruct((_B, _L, _D), jnp.float32),
    mesh=_mesh,
    scratch_types=[
        pltpu.VMEM((_LP,), jnp.int32),
        pltpu.VMEM((_LP,), jnp.int32),
        pltpu.VMEM((_LP,), jnp.int32),
        pltpu.VMEM((_LP,), jnp.int32),
        pltpu.VMEM((_LP, _D), jnp.float32),
        pltpu.VMEM((_LP, _D), jnp.float32),
        pltpu.VMEM((_LP, _D), jnp.float32),
        pltpu.VMEM((_LP, _D), jnp.float32),
        pltpu.VMEM_SHARED((_VOCAB, _D), jnp.float32),
        pltpu.SemaphoreType.DMA,
        pltpu.SemaphoreType.DMA,
        pltpu.SemaphoreType.DMA,
        pltpu.SemaphoreType.DMA,
        pltpu.SemaphoreType.DMA,
        pltpu.SemaphoreType.DMA,
        pltpu.SemaphoreType.DMA,
        pltpu.SemaphoreType.DMA,
    ],
    compiler_params=pltpu.CompilerParams(needs_layout_passes=False),
)
def _embed_kernel(ids_hbm, char_hbm, pos_hbm, out_hbm,
                  idx0, idx1, pidx0, pidx1, cbuf0, cbuf1, pbuf0, pbuf1,
                  ctab_s,
                  sem_i0, sem_i1, sem_g0, sem_g1, sem_p0, sem_p1,
                  sem_o0, sem_o1):
    idxs = (idx0, idx1)
    pidxs = (pidx0, pidx1)
    cbufs = (cbuf0, cbuf1)
    pbufs = (pbuf0, pbuf1)
    sem_i = (sem_i0, sem_i1)
    sem_g = (sem_g0, sem_g1)
    sem_p = (sem_p0, sem_p1)
    sem_o = (sem_o0, sem_o1)

    wid = lax.axis_index("s") * _NC + lax.axis_index("c")
    row0 = wid * _RW
    lanes = lax.iota(jnp.int32, 16)
    ones = jnp.full((16,), 1, jnp.int32)
    zeros = jnp.full((16,), 0, jnp.int32)

    @pl.when(lax.axis_index("s") == 0)
    def _stage_ctab():
        pltpu.sync_copy(char_hbm, ctab_s)

    plsc.subcore_barrier()

    def issue_ids(r, b):
        return pltpu.async_copy(ids_hbm.at[pl.ds(r * _L, _L)],
                                idxs[b].at[pl.ds(0, _L)], sem_i[b])

    def wait_ids(b):
        pltpu.make_async_copy(ids_hbm.at[pl.ds(0, _L)],
                              idxs[b].at[pl.ds(0, _L)], sem_i[b]).wait()

    def pos_and_gather(b):
        carry = zeros
        for j in range(_NCH):
            v = idxs[b][pl.ds(j * 16, 16)]
            if j == _NCH - 1:
                v = jnp.where(lanes < _TAIL, v, zeros)
                idxs[b][pl.ds(j * 16, 16)] = v
            m = jnp.where(v != 0, ones, zeros)
            pidxs[b][pl.ds(j * 16, 16)] = (plsc.cumsum(m) + carry) * m
            carry = carry + lax.broadcast_in_dim(jnp.sum(m), (16,), ())
        for lo, n in ((0, 128), (128, _LP - 128)):
            sl = pl.ds(lo, n)
            pltpu.async_copy(ctab_s.at[idxs[b].at[sl]], cbufs[b].at[sl],
                             sem_g[b])
            pltpu.async_copy(pos_hbm.at[pidxs[b].at[sl]], pbufs[b].at[sl],
                             sem_p[b])

    def wait_gathers(b):
        pltpu.make_async_copy(char_hbm.at[pl.ds(0, _LP)], cbufs[b],
                              sem_g[b]).wait()
        pltpu.make_async_copy(pos_hbm.at[pl.ds(0, _LP)], pbufs[b],
                              sem_p[b]).wait()

    def wait_out(b, row):
        pltpu.make_async_copy(cbufs[b].at[pl.ds(0, _L)],
                              out_hbm.at[row], sem_o[b]).wait()

    issue_ids(row0, 0).wait()
    issue_ids(row0 + 1, 1)
    pos_and_gather(0)

    def outer(o, acc):
        for par in range(2):
            i = o * 2 + par
            cur, nxt = par, 1 - par
            row = row0 + i

            @pl.when(i < _RW - 1)
            def _prep_next():
                wait_ids(nxt)

                @pl.when(i > 0)
                def _wait_prev_out():
                    wait_out(nxt, row)

                pos_and_gather(nxt)

            wait_gathers(cur)

            @pl.when(i < _RW - 2)
            def _issue_ids2():
                issue_ids(row + 2, cur)

            def add_body(r, a):
                for k in range(_D // 16):
                    sl = pl.ds(k * 16, 16)
                    plsc.addupdate(cbufs[cur].at[r, sl], pbufs[cur][r, sl])
                return a

            lax.fori_loop(0, _L, add_body, 0)
            pltpu.async_copy(cbufs[cur].at[pl.ds(0, _L)], out_hbm.at[row],
                             sem_o[cur])
        return acc

    lax.fori_loop(0, _RW // 2, outer, 0)
    wait_out(0, row0 + _RW - 2)
    wait_out(1, row0 + _RW - 1)


def kernel(input_ids, char_table, pos_table):
    return _embed_kernel(input_ids.reshape(-1), char_table, pos_table)


# speedup vs baseline: 9.9599x; 4.8592x over previous
import functools

import jax
import jax.numpy as jnp
from jax import lax
from jax.experimental import pallas as pl
from jax.experimental.pallas import tpu as pltpu
from jax.experimental.pallas import tpu_sc as plsc

_B, _L, _D = 1024, 200, 128
_NC, _NS = 2, 16
_NW = _NC * _NS
_RW = _B // _NW
_NCH = (_L + 15) // 16
_LP = _NCH * 16
_TAIL = _L - (_NCH - 1) * 16
_VOCAB = 1000

_mesh = plsc.VectorSubcoreMesh(core_axis_name="c", subcore_axis_name="s")


@functools.partial(
    pl.kernel,
    out_type=jax.ShapeDtypeStruct((_B, _L, _D), jnp.float32),
    mesh=_mesh,
    scratch_types=[
        pltpu.VMEM((_LP,), jnp.int32),
        pltpu.VMEM((_LP,), jnp.int32),
        pltpu.VMEM((_LP,), jnp.int32),
        pltpu.VMEM((_LP,), jnp.int32),
        pltpu.VMEM((_LP, _D), jnp.float32),
        pltpu.VMEM((_LP, _D), jnp.float32),
        pltpu.VMEM((_LP, _D), jnp.float32),
        pltpu.VMEM((_LP, _D), jnp.float32),
        pltpu.VMEM_SHARED((_VOCAB, _D), jnp.float32),
        pltpu.VMEM_SHARED((256, _D), jnp.float32),
        pltpu.SemaphoreType.DMA,
        pltpu.SemaphoreType.DMA,
        pltpu.SemaphoreType.DMA,
        pltpu.SemaphoreType.DMA,
        pltpu.SemaphoreType.DMA,
        pltpu.SemaphoreType.DMA,
        pltpu.SemaphoreType.DMA,
        pltpu.SemaphoreType.DMA,
    ],
    compiler_params=pltpu.CompilerParams(needs_layout_passes=False),
)
def _embed_kernel(ids_hbm, char_hbm, pos_hbm, out_hbm,
                  idx0, idx1, pidx0, pidx1, cbuf0, cbuf1, pbuf0, pbuf1,
                  ctab_s, ptab_s,
                  sem_i0, sem_i1, sem_g0, sem_g1, sem_p0, sem_p1,
                  sem_o0, sem_o1):
    idxs = (idx0, idx1)
    pidxs = (pidx0, pidx1)
    cbufs = (cbuf0, cbuf1)
    pbufs = (pbuf0, pbuf1)
    sem_i = (sem_i0, sem_i1)
    sem_g = (sem_g0, sem_g1)
    sem_p = (sem_p0, sem_p1)
    sem_o = (sem_o0, sem_o1)

    wid = lax.axis_index("s") * _NC + lax.axis_index("c")
    row0 = wid * _RW
    lanes = lax.iota(jnp.int32, 16)
    ones = jnp.full((16,), 1, jnp.int32)
    zeros = jnp.full((16,), 0, jnp.int32)

    @pl.when(lax.axis_index("s") == 0)
    def _stage_ctab():
        pltpu.sync_copy(char_hbm, ctab_s)

    @pl.when(lax.axis_index("s") == 1)
    def _stage_ptab():
        pltpu.sync_copy(pos_hbm, ptab_s)

    plsc.subcore_barrier()

    def issue_ids(r, b):
        return pltpu.async_copy(ids_hbm.at[pl.ds(r * _L, _L)],
                                idxs[b].at[pl.ds(0, _L)], sem_i[b])

    def wait_ids(b):
        pltpu.make_async_copy(ids_hbm.at[pl.ds(0, _L)],
                              idxs[b].at[pl.ds(0, _L)], sem_i[b]).wait()

    def pos_and_gather(b):
        carry = zeros
        for j in range(_NCH):
            v = idxs[b][pl.ds(j * 16, 16)]
            if j == _NCH - 1:
                v = jnp.where(lanes < _TAIL, v, zeros)
                idxs[b][pl.ds(j * 16, 16)] = v
            m = jnp.where(v != 0, ones, zeros)
            pidxs[b][pl.ds(j * 16, 16)] = (plsc.cumsum(m) + carry) * m
            carry = carry + lax.broadcast_in_dim(jnp.sum(m), (16,), ())
        for lo, n in ((0, 128), (128, _LP - 128)):
            sl = pl.ds(lo, n)
            pltpu.async_copy(ctab_s.at[idxs[b].at[sl]], cbufs[b].at[sl],
                             sem_g[b])
            pltpu.async_copy(ptab_s.at[pidxs[b].at[sl]], pbufs[b].at[sl],
                             sem_p[b])

    def wait_gathers(b):
        pltpu.make_async_copy(char_hbm.at[pl.ds(0, _LP)], cbufs[b],
                              sem_g[b]).wait()
        pltpu.make_async_copy(pos_hbm.at[pl.ds(0, _LP)], pbufs[b],
                              sem_p[b]).wait()

    def wait_out(b, row):
        pltpu.make_async_copy(cbufs[b].at[pl.ds(0, _L)],
                              out_hbm.at[row], sem_o[b]).wait()

    issue_ids(row0, 0).wait()
    issue_ids(row0 + 1, 1)
    pos_and_gather(0)

    def outer(o, acc):
        for par in range(2):
            i = o * 2 + par
            cur, nxt = par, 1 - par
            row = row0 + i

            @pl.when(i < _RW - 1)
            def _prep_next():
                wait_ids(nxt)

                @pl.when(i > 0)
                def _wait_prev_out():
                    wait_out(nxt, row)

                pos_and_gather(nxt)

            wait_gathers(cur)

            @pl.when(i < _RW - 2)
            def _issue_ids2():
                issue_ids(row + 2, cur)

            def add_body(r, a):
                for k in range(_D // 16):
                    sl = pl.ds(k * 16, 16)
                    plsc.addupdate(cbufs[cur].at[r, sl], pbufs[cur][r, sl])
                return a

            lax.fori_loop(0, _L, add_body, 0)
            pltpu.async_copy(cbufs[cur].at[pl.ds(0, _L)], out_hbm.at[row],
                             sem_o[cur])
        return acc

    lax.fori_loop(0, _RW // 2, outer, 0)
    wait_out(0, row0 + _RW - 2)
    wait_out(1, row0 + _RW - 1)


def kernel(input_ids, char_table, pos_table):
    return _embed_kernel(input_ids.reshape(-1), char_table, pos_table)
